# Initial kernel scaffold; baseline (speedup 1.0000x reference)
import jax
import jax.numpy as jnp
from jax.experimental import pallas as pl


def kernel(x, edge_index, W1, b1, W2, b2, W3, b3, p1, p2, p3):
    raise NotImplementedError("write your pallas kernel here")


# speedup vs baseline: 10.5157x; 10.5157x over previous
import functools
import math

import jax
import jax.numpy as jnp
from jax import lax
from jax.experimental import pallas as pl
from jax.experimental.pallas import tpu as pltpu
from jax.experimental.pallas import tpu_sc as plsc

N = 10000
E = 160000
H = 256
NP = 10240
EP = 163840
EPT = EP // 32
CHUNKS = 80
STRIPE = NP // 16
BR = 1280
GRID = NP // BR


def _mesh():
    return plsc.VectorSubcoreMesh(core_axis_name="c", subcore_axis_name="s")


NPR = NP // 128
DSTR = NPR // 16


def _sc_deg(srcf, dstf, alive, zeros2, rowidx):
    @functools.partial(
        pl.kernel,
        out_type=[jax.ShapeDtypeStruct((NPR, 128), jnp.float32),
                  jax.ShapeDtypeStruct((NPR, 128), jnp.float32)],
        mesh=_mesh(),
        compiler_params=pltpu.CompilerParams(needs_layout_passes=False),
        scratch_types=[
            pltpu.VMEM((NP,), jnp.float32),
            pltpu.VMEM((EPT,), jnp.int32),
            pltpu.VMEM((EPT,), jnp.int32),
            pltpu.VMEM((NPR, 128), jnp.float32),
            pltpu.VMEM((NPR,), jnp.int32),
            pltpu.VMEM_SHARED((NPR, 128), jnp.float32),
        ],
    )
    def k(src_h, dst_h, alive_h, z2_h, ri_h, deg0_h, deg1_h,
          alive_v, src_v, dst_v, deg_v, ridx_v, deg_s):
        c = lax.axis_index("c")
        s = lax.axis_index("s")
        base = (c * 16 + s) * EPT
        pltpu.sync_copy(src_h.at[pl.ds(base, EPT)], src_v)
        pltpu.sync_copy(dst_h.at[pl.ds(base, EPT)], dst_v)
        pltpu.sync_copy(alive_h, alive_v)
        pltpu.sync_copy(z2_h, deg_v)
        pltpu.sync_copy(ri_h, ridx_v)

        @pl.when(s == 0)
        def _():
            pltpu.sync_copy(z2_h, deg_s)

        plsc.subcore_barrier()

        @pl.loop(0, EPT // 16)
        def _(i):
            off = i * 16
            s16 = src_v[pl.ds(off, 16)]
            d16 = dst_v[pl.ds(off, 16)]
            a16 = plsc.load_gather(alive_v, [s16])
            row = lax.shift_right_logical(d16, 7)
            col = lax.bitwise_and(d16, 127)
            plsc.addupdate_scatter(deg_v, [row, col], a16)

        pltpu.sync_copy(deg_v, deg_s.at[ridx_v], add=True)
        plsc.subcore_barrier()

        @pl.when((c == 0) & (s < NPR // 8))
        def _():
            pltpu.sync_copy(deg_s.at[pl.ds(s * 8, 8)],
                            deg0_h.at[pl.ds(s * 8, 8)])

        @pl.when((c == 1) & (s < NPR // 8))
        def _():
            pltpu.sync_copy(deg_s.at[pl.ds(s * 8, 8)],
                            deg1_h.at[pl.ds(s * 8, 8)])

    return k(srcf, dstf, alive, zeros2, rowidx)


def _sc_agg(g0, g1, src2d, dst2d, zcols):
    @functools.partial(
        pl.kernel,
        out_type=[jax.ShapeDtypeStruct((NP, 128), jnp.float32),
                  jax.ShapeDtypeStruct((NP, 128), jnp.float32)],
        mesh=_mesh(),
        compiler_params=pltpu.CompilerParams(needs_layout_passes=False),
        scratch_types=[
            pltpu.VMEM((CHUNKS, 128), jnp.int32),
            pltpu.VMEM((CHUNKS, 128), jnp.int32),
            pltpu.VMEM((128, 128), jnp.float32),
            pltpu.VMEM_SHARED((NP, 128), jnp.float32),
            pltpu.SemaphoreType.DMA,
        ],
    )
    def k(g0_h, g1_h, s2_h, d2_h, zc_h, a0_h, a1_h,
          src_v, dst_v, rows_v, acc, sem):
        c = lax.axis_index("c")
        s = lax.axis_index("s")
        pltpu.sync_copy(s2_h.at[pl.ds(s * CHUNKS, CHUNKS)], src_v)
        pltpu.sync_copy(d2_h.at[pl.ds(s * CHUNKS, CHUNKS)], dst_v)
        pltpu.sync_copy(zc_h, acc.at[pl.ds(s * STRIPE, STRIPE)])
        plsc.subcore_barrier()

        def run(g_h):
            @pl.loop(0, CHUNKS)
            def _(i):
                pltpu.async_copy(g_h.at[src_v.at[i]], rows_v, sem).wait()
                pltpu.sync_copy(rows_v, acc.at[dst_v.at[i]], add=True)

        @pl.when(c == 0)
        def _():
            run(g0_h)

        @pl.when(c == 1)
        def _():
            run(g1_h)

        plsc.subcore_barrier()

        @pl.when(c == 0)
        def _():
            pltpu.sync_copy(acc.at[pl.ds(s * STRIPE, STRIPE)],
                            a0_h.at[pl.ds(s * STRIPE, STRIPE)])

        @pl.when(c == 1)
        def _():
            pltpu.sync_copy(acc.at[pl.ds(s * STRIPE, STRIPE)],
                            a1_h.at[pl.ds(s * STRIPE, STRIPE)])

    return k(g0, g1, src2d, dst2d, zcols)


def _k1_body(x_ref, w_ref, d0_ref, d1_ref, al_ref, g0_ref, g1_ref, de_ref):
    de = lax.rsqrt(1.0 + d0_ref[...] + d1_ref[...]) * al_ref[...]
    h = jnp.dot(x_ref[...], w_ref[...], preferred_element_type=jnp.float32)
    g = h * de
    g0_ref[...] = g[:, :128]
    g1_ref[...] = g[:, 128:]
    de_ref[...] = de


def _tc_k1(x_p, W, deg0, deg1, alive_c):
    return pl.pallas_call(
        _k1_body,
        grid=(GRID,),
        in_specs=[
            pl.BlockSpec((BR, H), lambda b: (b, 0)),
            pl.BlockSpec((H, H), lambda b: (0, 0)),
            pl.BlockSpec((BR, 1), lambda b: (b, 0)),
            pl.BlockSpec((BR, 1), lambda b: (b, 0)),
            pl.BlockSpec((BR, 1), lambda b: (b, 0)),
        ],
        out_specs=[
            pl.BlockSpec((BR, 128), lambda b: (b, 0)),
            pl.BlockSpec((BR, 128), lambda b: (b, 0)),
            pl.BlockSpec((BR, 1), lambda b: (b, 0)),
        ],
        out_shape=[
            jax.ShapeDtypeStruct((NP, 128), jnp.float32),
            jax.ShapeDtypeStruct((NP, 128), jnp.float32),
            jax.ShapeDtypeStruct((NP, 1), jnp.float32),
        ],
    )(x_p, W, deg0, deg1, alive_c)


def _k2_body(a0_ref, a1_ref, g0_ref, g1_ref, de_ref, b_ref, p_ref,
             xn_ref, z_ref):
    u = jnp.concatenate([a0_ref[...] + g0_ref[...],
                         a1_ref[...] + g1_ref[...]], axis=1)
    pre = u * de_ref[...] + b_ref[...]
    xn = jnp.where(pre > 0, pre, jnp.exp(jnp.minimum(pre, 0.0)) - 1.0)
    xn_ref[...] = xn
    z_ref[...] = lax.dot_general(xn, p_ref[...], (((1,), (1,)), ((), ())),
                                 preferred_element_type=jnp.float32)


def _tc_k2(agg0, agg1, g0, g1, de, b2d, p2d):
    return pl.pallas_call(
        _k2_body,
        grid=(GRID,),
        in_specs=[
            pl.BlockSpec((BR, 128), lambda b: (b, 0)),
            pl.BlockSpec((BR, 128), lambda b: (b, 0)),
            pl.BlockSpec((BR, 128), lambda b: (b, 0)),
            pl.BlockSpec((BR, 128), lambda b: (b, 0)),
            pl.BlockSpec((BR, 1), lambda b: (b, 0)),
            pl.BlockSpec((1, H), lambda b: (0, 0)),
            pl.BlockSpec((1, H), lambda b: (0, 0)),
        ],
        out_specs=[
            pl.BlockSpec((BR, H), lambda b: (b, 0)),
            pl.BlockSpec((BR, 1), lambda b: (b, 0)),
        ],
        out_shape=[
            jax.ShapeDtypeStruct((NP, H), jnp.float32),
            jax.ShapeDtypeStruct((NP, 1), jnp.float32),
        ],
    )(agg0, agg1, g0, g1, de, b2d, p2d)


def _make_k3_body(kk):
    def body(z_ref, al_ref, p_ref, kept_ref, sk_ref):
        zb = z_ref[...]
        al = al_ref[...]
        bits = lax.bitcast_convert_type(zb, jnp.int32)
        key = jnp.where(bits >= 0, bits, bits ^ jnp.int32(0x7FFFFFFF))
        ukey = lax.bitcast_convert_type(key ^ jnp.int32(-2147483648),
                                        jnp.uint32)
        ukey = jnp.where(al > 0, ukey, jnp.uint32(0))
        t = jnp.uint32(0)
        for bit in range(31, -1, -1):
            cand = t | jnp.uint32(1 << bit)
            cnt = jnp.sum((ukey >= cand).astype(jnp.int32))
            t = jnp.where(cnt >= kk, cand, t)
        cnt_gt = jnp.sum((ukey > t).astype(jnp.int32))
        m = kk - cnt_gt
        tie = (ukey == t) & (al > 0)
        idx = lax.broadcasted_iota(jnp.int32, (1, NP), 1)
        j = jnp.int32(0)
        for bit in range(13, -1, -1):
            cand = j | jnp.int32(1 << bit)
            cntt = jnp.sum((tie & (idx < cand)).astype(jnp.int32))
            j = jnp.where(cntt < m, cand, j)
        kept = (ukey > t) | (tie & (idx < j + 1))
        keptf = kept.astype(jnp.float32)
        sc = jnp.tanh(zb * lax.rsqrt(jnp.sum(p_ref[...] * p_ref[...])))
        kept_ref[...] = keptf
        sk_ref[...] = sc * keptf
    return body


def _tc_k3(z_row, alive_row, p2d, kk):
    return pl.pallas_call(
        _make_k3_body(kk),
        out_shape=[
            jax.ShapeDtypeStruct((1, NP), jnp.float32),
            jax.ShapeDtypeStruct((1, NP), jnp.float32),
        ],
    )(z_row, alive_row, p2d)


def _k4_body(xn_ref, sk_ref, xo_ref, ps_ref):
    xo = xn_ref[...] * sk_ref[...]
    xo_ref[...] = xo
    ps_ref[...] = jnp.sum(xo, axis=0, keepdims=True)[None]


def _tc_k4(xn, sk_col):
    return pl.pallas_call(
        _k4_body,
        grid=(GRID,),
        in_specs=[
            pl.BlockSpec((BR, H), lambda b: (b, 0)),
            pl.BlockSpec((BR, 1), lambda b: (b, 0)),
        ],
        out_specs=[
            pl.BlockSpec((BR, H), lambda b: (b, 0)),
            pl.BlockSpec((1, 1, H), lambda b: (b, 0, 0)),
        ],
        out_shape=[
            jax.ShapeDtypeStruct((NP, H), jnp.float32),
            jax.ShapeDtypeStruct((GRID, 1, H), jnp.float32),
        ],
    )(xn, sk_col)


def kernel(x, edge_index, W1, b1, W2, b2, W3, b3, p1, p2, p3):
    i32 = jnp.int32
    f32 = jnp.float32
    src = edge_index[0].astype(i32)
    dst = edge_index[1].astype(i32)
    pad = jnp.full((EP - E,), N, i32)
    srcf = jnp.concatenate([src, pad])
    dstf = jnp.concatenate([dst, pad])
    src2d = srcf.reshape(EP // 128, 128)
    dst2d = dstf.reshape(EP // 128, 128)
    zeros2 = jnp.zeros((NPR, 128), f32)
    rowidx = jnp.arange(NPR, dtype=i32)
    zcols = jnp.zeros((STRIPE, 128), f32)
    x_p = jnp.pad(x, ((0, NP - N), (0, 0)))
    alive = jnp.concatenate([jnp.ones((N,), f32), jnp.zeros((NP - N,), f32)])

    outs = []
    kk = N
    for (W, b, p) in ((W1, b1, p1), (W2, b2, p2), (W3, b3, p3)):
        kk = int(math.ceil(0.5 * kk))
        deg0, deg1 = _sc_deg(srcf, dstf, alive, zeros2, rowidx)
        g0, g1, de = _tc_k1(x_p, W, deg0.reshape(NP, 1), deg1.reshape(NP, 1),
                            alive.reshape(NP, 1))
        agg0, agg1 = _sc_agg(g0, g1, src2d, dst2d, zcols)
        xn, z = _tc_k2(agg0, agg1, g0, g1, de, b.reshape(1, H),
                       p.reshape(1, H))
        kept, sk = _tc_k3(z.reshape(1, NP), alive.reshape(1, NP),
                          p.reshape(1, H), kk)
        x_p, psum = _tc_k4(xn, sk.reshape(NP, 1))
        outs.append(jnp.sum(psum, axis=0) / kk)
        alive = kept.reshape(NP)
    return tuple(outs)


# speedup vs baseline: 12.5608x; 1.1945x over previous
import functools
import math

import jax
import jax.numpy as jnp
from jax import lax
from jax.experimental import pallas as pl
from jax.experimental.pallas import tpu as pltpu
from jax.experimental.pallas import tpu_sc as plsc

N = 10000
E = 160000
H = 256
NP = 10240
EP = 163840
EPT = EP // 32
CH = 64
CHUNKS = 160
NBUF = 2
STRIPE = NP // 16
BR = 1280
GRID = NP // BR


def _mesh():
    return plsc.VectorSubcoreMesh(core_axis_name="c", subcore_axis_name="s")


NPR = NP // 128
DSTR = NPR // 16


def _sc_deg(srcf, dstf, alive, zeros2, rowidx):
    @functools.partial(
        pl.kernel,
        out_type=[jax.ShapeDtypeStruct((NPR, 128), jnp.float32),
                  jax.ShapeDtypeStruct((NPR, 128), jnp.float32)],
        mesh=_mesh(),
        compiler_params=pltpu.CompilerParams(needs_layout_passes=False),
        scratch_types=[
            pltpu.VMEM((NP,), jnp.float32),
            pltpu.VMEM((EPT,), jnp.int32),
            pltpu.VMEM((EPT,), jnp.int32),
            pltpu.VMEM((NPR, 128), jnp.float32),
            pltpu.VMEM((NPR,), jnp.int32),
            pltpu.VMEM_SHARED((NPR, 128), jnp.float32),
        ],
    )
    def k(src_h, dst_h, alive_h, z2_h, ri_h, deg0_h, deg1_h,
          alive_v, src_v, dst_v, deg_v, ridx_v, deg_s):
        c = lax.axis_index("c")
        s = lax.axis_index("s")
        base = (c * 16 + s) * EPT
        pltpu.sync_copy(src_h.at[pl.ds(base, EPT)], src_v)
        pltpu.sync_copy(dst_h.at[pl.ds(base, EPT)], dst_v)
        pltpu.sync_copy(alive_h, alive_v)
        pltpu.sync_copy(z2_h, deg_v)
        pltpu.sync_copy(ri_h, ridx_v)

        @pl.when(s == 0)
        def _():
            pltpu.sync_copy(z2_h, deg_s)

        plsc.subcore_barrier()

        @pl.loop(0, EPT // 16)
        def _(i):
            off = i * 16
            s16 = src_v[pl.ds(off, 16)]
            d16 = dst_v[pl.ds(off, 16)]
            a16 = plsc.load_gather(alive_v, [s16])
            row = lax.shift_right_logical(d16, 7)
            col = lax.bitwise_and(d16, 127)
            plsc.addupdate_scatter(deg_v, [row, col], a16)

        pltpu.sync_copy(deg_v, deg_s.at[ridx_v], add=True)
        plsc.subcore_barrier()

        @pl.when((c == 0) & (s < NPR // 8))
        def _():
            pltpu.sync_copy(deg_s.at[pl.ds(s * 8, 8)],
                            deg0_h.at[pl.ds(s * 8, 8)])

        @pl.when((c == 1) & (s < NPR // 8))
        def _():
            pltpu.sync_copy(deg_s.at[pl.ds(s * 8, 8)],
                            deg1_h.at[pl.ds(s * 8, 8)])

    return k(srcf, dstf, alive, zeros2, rowidx)


def _sc_agg(g0, g1, src2d, dst2d, zcols):
    @functools.partial(
        pl.kernel,
        out_type=[jax.ShapeDtypeStruct((NP, 128), jnp.float32),
                  jax.ShapeDtypeStruct((NP, 128), jnp.float32)],
        mesh=_mesh(),
        compiler_params=pltpu.CompilerParams(needs_layout_passes=False),
        scratch_types=[
            pltpu.VMEM((CHUNKS // 2, CH), jnp.int32),
            pltpu.VMEM((CHUNKS // 2, CH), jnp.int32),
            pltpu.VMEM((NBUF, CH, 128), jnp.float32),
            pltpu.VMEM_SHARED((NP, 128), jnp.float32),
            [pltpu.SemaphoreType.DMA] * NBUF,
            [pltpu.SemaphoreType.DMA] * NBUF,
        ],
    )
    def k(g0_h, g1_h, s2_h, d2_h, zc_h, a0_h, a1_h,
          src_v, dst_v, rows_v, acc, gsem, ssem):
        c = lax.axis_index("c")
        s = lax.axis_index("s")
        half = CHUNKS // 2
        pltpu.sync_copy(zc_h, acc.at[pl.ds(s * STRIPE, STRIPE)])
        plsc.subcore_barrier()

        def run(g_h):
            def start_gather(j, b):
                pltpu.async_copy(g_h.at[src_v.at[j]], rows_v.at[b], gsem[b])

            for p in range(2):
                base = s * CHUNKS + p * half
                pltpu.sync_copy(s2_h.at[pl.ds(base, half)], src_v)
                pltpu.sync_copy(d2_h.at[pl.ds(base, half)], dst_v)
                for b in range(NBUF):
                    start_gather(b, b)

                @pl.loop(0, half, step=NBUF)
                def _(i):
                    for b in range(NBUF):
                        j = i + b
                        pltpu.make_async_copy(g_h.at[src_v.at[0]],
                                              rows_v.at[b], gsem[b]).wait()
                        pltpu.async_copy(rows_v.at[b], acc.at[dst_v.at[j]],
                                         ssem[b], add=True)

                        @pl.when(j + NBUF < half)
                        def _():
                            pltpu.make_async_copy(rows_v.at[b],
                                                  acc.at[dst_v.at[0]],
                                                  ssem[b]).wait()
                            start_gather(j + NBUF, b)

                for b in range(NBUF):
                    pltpu.make_async_copy(rows_v.at[b], acc.at[dst_v.at[0]],
                                          ssem[b]).wait()

        @pl.when(c == 0)
        def _():
            run(g0_h)

        @pl.when(c == 1)
        def _():
            run(g1_h)

        plsc.subcore_barrier()

        @pl.when(c == 0)
        def _():
            pltpu.sync_copy(acc.at[pl.ds(s * STRIPE, STRIPE)],
                            a0_h.at[pl.ds(s * STRIPE, STRIPE)])

        @pl.when(c == 1)
        def _():
            pltpu.sync_copy(acc.at[pl.ds(s * STRIPE, STRIPE)],
                            a1_h.at[pl.ds(s * STRIPE, STRIPE)])

    return k(g0, g1, src2d, dst2d, zcols)


def _k1_body(x_ref, w_ref, d0_ref, d1_ref, al_ref, g0_ref, g1_ref, de_ref):
    de = lax.rsqrt(1.0 + d0_ref[...] + d1_ref[...]) * al_ref[...]
    h = jnp.dot(x_ref[...], w_ref[...], preferred_element_type=jnp.float32)
    g = h * de
    g0_ref[...] = g[:, :128]
    g1_ref[...] = g[:, 128:]
    de_ref[...] = de


def _tc_k1(x_p, W, deg0, deg1, alive_c):
    return pl.pallas_call(
        _k1_body,
        grid=(GRID,),
        in_specs=[
            pl.BlockSpec((BR, H), lambda b: (b, 0)),
            pl.BlockSpec((H, H), lambda b: (0, 0)),
            pl.BlockSpec((BR, 1), lambda b: (b, 0)),
            pl.BlockSpec((BR, 1), lambda b: (b, 0)),
            pl.BlockSpec((BR, 1), lambda b: (b, 0)),
        ],
        out_specs=[
            pl.BlockSpec((BR, 128), lambda b: (b, 0)),
            pl.BlockSpec((BR, 128), lambda b: (b, 0)),
            pl.BlockSpec((BR, 1), lambda b: (b, 0)),
        ],
        out_shape=[
            jax.ShapeDtypeStruct((NP, 128), jnp.float32),
            jax.ShapeDtypeStruct((NP, 128), jnp.float32),
            jax.ShapeDtypeStruct((NP, 1), jnp.float32),
        ],
    )(x_p, W, deg0, deg1, alive_c)


def _k2_body(a0_ref, a1_ref, g0_ref, g1_ref, de_ref, b_ref, p_ref,
             xn_ref, z_ref):
    u = jnp.concatenate([a0_ref[...] + g0_ref[...],
                         a1_ref[...] + g1_ref[...]], axis=1)
    pre = u * de_ref[...] + b_ref[...]
    xn = jnp.where(pre > 0, pre, jnp.exp(jnp.minimum(pre, 0.0)) - 1.0)
    xn_ref[...] = xn
    z_ref[...] = lax.dot_general(xn, p_ref[...], (((1,), (1,)), ((), ())),
                                 preferred_element_type=jnp.float32)


def _tc_k2(agg0, agg1, g0, g1, de, b2d, p2d):
    return pl.pallas_call(
        _k2_body,
        grid=(GRID,),
        in_specs=[
            pl.BlockSpec((BR, 128), lambda b: (b, 0)),
            pl.BlockSpec((BR, 128), lambda b: (b, 0)),
            pl.BlockSpec((BR, 128), lambda b: (b, 0)),
            pl.BlockSpec((BR, 128), lambda b: (b, 0)),
            pl.BlockSpec((BR, 1), lambda b: (b, 0)),
            pl.BlockSpec((1, H), lambda b: (0, 0)),
            pl.BlockSpec((1, H), lambda b: (0, 0)),
        ],
        out_specs=[
            pl.BlockSpec((BR, H), lambda b: (b, 0)),
            pl.BlockSpec((BR, 1), lambda b: (b, 0)),
        ],
        out_shape=[
            jax.ShapeDtypeStruct((NP, H), jnp.float32),
            jax.ShapeDtypeStruct((NP, 1), jnp.float32),
        ],
    )(agg0, agg1, g0, g1, de, b2d, p2d)


def _make_k3_body(kk):
    def body(z_ref, al_ref, p_ref, kept_ref, sk_ref):
        zb = z_ref[...]
        al = al_ref[...]
        bits = lax.bitcast_convert_type(zb, jnp.int32)
        key = jnp.where(bits >= 0, bits, bits ^ jnp.int32(0x7FFFFFFF))
        ukey = lax.bitcast_convert_type(key ^ jnp.int32(-2147483648),
                                        jnp.uint32)
        ukey = jnp.where(al > 0, ukey, jnp.uint32(0))
        t = jnp.uint32(0)
        for bit in range(31, -1, -1):
            cand = t | jnp.uint32(1 << bit)
            cnt = jnp.sum((ukey >= cand).astype(jnp.int32))
            t = jnp.where(cnt >= kk, cand, t)
        cnt_gt = jnp.sum((ukey > t).astype(jnp.int32))
        m = kk - cnt_gt
        tie = (ukey == t) & (al > 0)
        idx = lax.broadcasted_iota(jnp.int32, (1, NP), 1)
        j = jnp.int32(0)
        for bit in range(13, -1, -1):
            cand = j | jnp.int32(1 << bit)
            cntt = jnp.sum((tie & (idx < cand)).astype(jnp.int32))
            j = jnp.where(cntt < m, cand, j)
        kept = (ukey > t) | (tie & (idx < j + 1))
        keptf = kept.astype(jnp.float32)
        sc = jnp.tanh(zb * lax.rsqrt(jnp.sum(p_ref[...] * p_ref[...])))
        kept_ref[...] = keptf
        sk_ref[...] = sc * keptf
    return body


def _tc_k3(z_row, alive_row, p2d, kk):
    return pl.pallas_call(
        _make_k3_body(kk),
        out_shape=[
            jax.ShapeDtypeStruct((1, NP), jnp.float32),
            jax.ShapeDtypeStruct((1, NP), jnp.float32),
        ],
    )(z_row, alive_row, p2d)


def _k4_body(xn_ref, sk_ref, xo_ref, ps_ref):
    xo = xn_ref[...] * sk_ref[...]
    xo_ref[...] = xo
    ps_ref[...] = jnp.sum(xo, axis=0, keepdims=True)[None]


def _tc_k4(xn, sk_col):
    return pl.pallas_call(
        _k4_body,
        grid=(GRID,),
        in_specs=[
            pl.BlockSpec((BR, H), lambda b: (b, 0)),
            pl.BlockSpec((BR, 1), lambda b: (b, 0)),
        ],
        out_specs=[
            pl.BlockSpec((BR, H), lambda b: (b, 0)),
            pl.BlockSpec((1, 1, H), lambda b: (b, 0, 0)),
        ],
        out_shape=[
            jax.ShapeDtypeStruct((NP, H), jnp.float32),
            jax.ShapeDtypeStruct((GRID, 1, H), jnp.float32),
        ],
    )(xn, sk_col)


def kernel(x, edge_index, W1, b1, W2, b2, W3, b3, p1, p2, p3):
    i32 = jnp.int32
    f32 = jnp.float32
    src = edge_index[0].astype(i32)
    dst = edge_index[1].astype(i32)
    pad = jnp.full((EP - E,), N, i32)
    srcf = jnp.concatenate([src, pad])
    dstf = jnp.concatenate([dst, pad])
    src2d = srcf.reshape(EP // CH, CH)
    dst2d = dstf.reshape(EP // CH, CH)
    zeros2 = jnp.zeros((NPR, 128), f32)
    rowidx = jnp.arange(NPR, dtype=i32)
    zcols = jnp.zeros((STRIPE, 128), f32)
    x_p = jnp.pad(x, ((0, NP - N), (0, 0)))
    alive = jnp.concatenate([jnp.ones((N,), f32), jnp.zeros((NP - N,), f32)])

    outs = []
    kk = N
    for (W, b, p) in ((W1, b1, p1), (W2, b2, p2), (W3, b3, p3)):
        kk = int(math.ceil(0.5 * kk))
        deg0, deg1 = _sc_deg(srcf, dstf, alive, zeros2, rowidx)
        g0, g1, de = _tc_k1(x_p, W, deg0.reshape(NP, 1), deg1.reshape(NP, 1),
                            alive.reshape(NP, 1))
        agg0, agg1 = _sc_agg(g0, g1, src2d, dst2d, zcols)
        xn, z = _tc_k2(agg0, agg1, g0, g1, de, b.reshape(1, H),
                       p.reshape(1, H))
        kept, sk = _tc_k3(z.reshape(1, NP), alive.reshape(1, NP),
                          p.reshape(1, H), kk)
        x_p, psum = _tc_k4(xn, sk.reshape(NP, 1))
        outs.append(jnp.sum(psum, axis=0) / kk)
        alive = kept.reshape(NP)
    return tuple(outs)
